# baseline (speedup 1.0000x reference)
import functools
import math

import jax
import jax.numpy as jnp
from jax import lax
from jax.experimental import pallas as pl
from jax.experimental.pallas import tpu as pltpu
from jax.experimental.pallas import tpu_sc as plsc

D_MODEL = 64
SCALE = math.sqrt(D_MODEL)

NC = 2
NS = 16
NW = NC * NS
LANES = 16

CHUNK = 128
NBUF = 4


def _sc_embed(x_flat, lut, *, n_rows):
    n_per_w = n_rows // NW
    n_chunks = n_per_w // CHUNK
    assert n_chunks % NBUF == 0

    mesh = plsc.VectorSubcoreMesh(core_axis_name="c", subcore_axis_name="s")

    @functools.partial(
        pl.kernel,
        mesh=mesh,
        out_type=jax.ShapeDtypeStruct((n_rows, D_MODEL), jnp.float32),
        scratch_types=[
            pltpu.VMEM((n_per_w,), jnp.int32),
            pltpu.VMEM((NBUF, CHUNK, D_MODEL), jnp.float32),
            pltpu.VMEM((NBUF, CHUNK, D_MODEL), jnp.float32),
            pltpu.SemaphoreType.DMA((NBUF,)),
            pltpu.SemaphoreType.DMA((NBUF,)),
        ],
        compiler_params=pltpu.CompilerParams(use_tc_tiling_on_sc=False),
    )
    def k(lut_hbm, idx_hbm, out_hbm, idx_v, raw_v, out_v, gsem, ssem):
        wid = lax.axis_index("s") * NC + lax.axis_index("c")
        base = wid * n_per_w
        pltpu.sync_copy(idx_hbm.at[pl.ds(base, n_per_w)], idx_v)

        def gather_start(c, b):
            pltpu.make_async_copy(
                lut_hbm.at[idx_v.at[pl.ds(c * CHUNK, CHUNK)]],
                raw_v.at[b], gsem.at[b],
            ).start()

        def gather_wait(c, b):
            pltpu.make_async_copy(
                lut_hbm.at[idx_v.at[pl.ds(c * CHUNK, CHUNK)]],
                raw_v.at[b], gsem.at[b],
            ).wait()

        def store_start(c, b):
            pltpu.make_async_copy(
                out_v.at[b], out_hbm.at[pl.ds(base + c * CHUNK, CHUNK)], ssem.at[b]
            ).start()

        def store_wait(b):
            pltpu.make_async_copy(
                out_v.at[b], out_hbm.at[pl.ds(base, CHUNK)], ssem.at[b]
            ).wait()

        for b in range(NBUF - 1):
            gather_start(b, b)

        def group_body(g, _):
            for b in range(NBUF):
                c = g * NBUF + b

                @pl.when(c + NBUF - 1 < n_chunks)
                def _(c=c, b=b):
                    gather_start(c + NBUF - 1, (b + NBUF - 1) % NBUF)

                gather_wait(c, b)

                @pl.when(c >= NBUF)
                def _(b=b):
                    store_wait(b)

                @plsc.parallel_loop(0, CHUNK, step=1, unroll=4)
                def _(i, b=b):
                    for j in range(D_MODEL // LANES):
                        s = pl.ds(j * LANES, LANES)
                        out_v[b, i, s] = raw_v[b, i, s] * SCALE

                store_start(c, b)
            return ()

        lax.fori_loop(0, n_chunks // NBUF, group_body, ())

        for b in range(NBUF):
            store_wait(b)

    return k(lut, x_flat)


def kernel(x, lut):
    b, s = x.shape
    x_flat = x.reshape(b * s).astype(jnp.int32)
    out = _sc_embed(x_flat, lut, n_rows=b * s)
    return out.reshape(b, s, D_MODEL)
